# Initial kernel scaffold; baseline (speedup 1.0000x reference)
import jax
import jax.numpy as jnp
from jax.experimental import pallas as pl


def kernel(point_cls_preds, img_seg_logits, voxels_2d, batch_idx):
    raise NotImplementedError("write your pallas kernel here")


# speedup vs baseline: 8.0699x; 8.0699x over previous
import jax
import jax.numpy as jnp
from jax import lax
from jax.experimental import pallas as pl
from jax.experimental.pallas import tpu as pltpu
from jax.experimental.pallas import tpu_sc as plsc

B = 4
NH, NW = 96, 320
RATIO = 4
H, W = NH * RATIO, NW * RATIO
IMG = B * NH * NW
CONF_THRES = 0.2

NUM_TILES = 32
CHUNK = 12544
PN = NUM_TILES * CHUNK
NCH = CHUNK // 128
ZSPAN = 7696
IMG_PAD = 16 * ZSPAN
DUMP = IMG


def _sc_scatter_body(xs_hbm, ys_hbm, pr_hbm, bt_hbm, out_hbm,
                     xs_v, ys_v, pr_v, bt_v, idx_v, val_v, img_sh):
    c = lax.axis_index("c")
    s = lax.axis_index("s")
    chunk_id = c * 16 + s
    base = chunk_id * CHUNK

    zeros16 = jnp.zeros((16,), jnp.float32)

    def zbody(i, carry):
        xs_v[pl.ds(i * 16, 16)] = zeros16
        return carry

    lax.fori_loop(0, ZSPAN // 16, zbody, 0)
    pltpu.sync_copy(xs_v.at[pl.ds(0, ZSPAN)], img_sh.at[pl.ds(s * ZSPAN, ZSPAN)])
    plsc.subcore_barrier()

    pltpu.sync_copy(xs_hbm.at[pl.ds(base, CHUNK)], xs_v)
    pltpu.sync_copy(ys_hbm.at[pl.ds(base, CHUNK)], ys_v)
    pltpu.sync_copy(pr_hbm.at[pl.ds(base, CHUNK)], pr_v)
    pltpu.sync_copy(bt_hbm.at[pl.ds(base, CHUNK)], bt_v)

    w_f = jnp.float32(W)
    h_f = jnp.float32(H)
    nw_f = jnp.float32(NW)
    nh_f = jnp.float32(NH)

    def cbody(j, carry):
        for k in range(8):
            o = j * 128 + k * 16
            x = xs_v[pl.ds(o, 16)]
            y = ys_v[pl.ds(o, 16)]
            p = pr_v[pl.ds(o, 16)]
            b = bt_v[pl.ds(o, 16)]
            nx = x / w_f
            ny = y / h_f
            valid = (nx >= 0.0) & (nx < 1.0) & (ny >= 0.0) & (ny < 1.0)
            ix = (nx * nw_f).astype(jnp.int32)
            iy = (ny * nh_f).astype(jnp.int32)
            flat = b * (NH * NW) + iy * NW + ix
            flat = jnp.where(valid, flat, DUMP)
            val = 1.0 / (1.0 + jnp.exp(-p))
            idx_v[j, pl.ds(k * 16, 16)] = flat
            val_v[j, pl.ds(k * 16, 16)] = val
        return carry

    lax.fori_loop(0, NCH, cbody, 0)

    def sbody(j, carry):
        pltpu.sync_copy(val_v.at[j], img_sh.at[idx_v.at[j]])
        return carry

    for r in range(16):
        @pl.when(s == r)
        def _():
            lax.fori_loop(0, NCH, sbody, 0)

        plsc.subcore_barrier()

    @pl.when(s == 0)
    def _():
        pltpu.sync_copy(img_sh.at[pl.ds(0, IMG)], out_hbm.at[c])


@jax.jit
def _sc_scatter(xs, ys, pr, bt):
    mesh = plsc.VectorSubcoreMesh(core_axis_name="c", subcore_axis_name="s")
    fn = pl.kernel(
        _sc_scatter_body,
        out_type=jax.ShapeDtypeStruct((2, IMG), jnp.float32),
        mesh=mesh,
        scratch_types=[
            pltpu.VMEM((CHUNK,), jnp.float32),
            pltpu.VMEM((CHUNK,), jnp.float32),
            pltpu.VMEM((CHUNK,), jnp.float32),
            pltpu.VMEM((CHUNK,), jnp.int32),
            pltpu.VMEM((NCH, 128), jnp.int32),
            pltpu.VMEM((NCH, 128), jnp.float32),
            pltpu.VMEM_SHARED((IMG_PAD,), jnp.float32),
        ],
    )
    return fn(xs, ys, pr, bt)


def _bce(p, t):
    logp = jnp.clip(jnp.log(p), -100.0)
    log1mp = jnp.clip(jnp.log(1.0 - p), -100.0)
    return -(t * logp + (1.0 - t) * log1mp)


def _loss_body(img0_ref, img1_ref, lg_ref, out_ref):
    i0 = img0_ref[...]
    i1 = img1_ref[...]
    lg = lg_ref[...]
    proj = jnp.where(i1 != 0.0, i1, i0)
    aux = jax.nn.sigmoid(lg)
    nz = proj != 0.0
    mask = nz & ((proj > CONF_THRES) | (aux > CONF_THRES))
    cnt = jnp.sum(mask.astype(jnp.float32))
    p = jnp.where(mask, proj, 0.5)
    t = jnp.where(mask, aux, 0.5)
    per_elem = (_bce(p, t) + _bce(t, p)) * 0.5
    total = jnp.sum(jnp.where(mask, per_elem, 0.0))
    out_ref[0, 0] = total / cnt / jnp.float32(B)


@jax.jit
def _tc_loss(img0, img1, lg):
    out = pl.pallas_call(
        _loss_body,
        out_shape=jax.ShapeDtypeStruct((1, 1), jnp.float32),
        out_specs=pl.BlockSpec(memory_space=pltpu.SMEM),
    )(img0, img1, lg)
    return out[0, 0]


def kernel(point_cls_preds, img_seg_logits, voxels_2d, batch_idx):
    n = voxels_2d.shape[0]
    pad = PN - n
    xs = jnp.concatenate([voxels_2d[:, 0], jnp.full((pad,), -1.0, jnp.float32)])
    ys = jnp.concatenate([voxels_2d[:, 1], jnp.full((pad,), -1.0, jnp.float32)])
    pr = jnp.concatenate([point_cls_preds[:, 0], jnp.zeros((pad,), jnp.float32)])
    bt = jnp.concatenate([batch_idx, jnp.zeros((pad,), jnp.int32)])
    imgs = _sc_scatter(xs, ys, pr, bt)
    img0 = imgs[0].reshape(960, 128)
    img1 = imgs[1].reshape(960, 128)
    lg = img_seg_logits.reshape(960, 128)
    return _tc_loss(img0, img1, lg)
